# baseline (speedup 1.0000x reference)
import jax
import jax.numpy as jnp
from jax import lax
from jax.experimental import pallas as pl
from jax.experimental.pallas import tpu as pltpu
from jax.experimental.pallas import tpu_sc as plsc

VOCAB = 100000
EMB = 128
OUT = 64
B = 4096
F = 26

NC, NS = 2, 16
NW = NC * NS
TOK = B * F
CHUNK = 128
NSLICE = 2
TOK_S = TOK // NSLICE
N_CHUNK = TOK_S // (NW * CHUNK)
PER_W = N_CHUNK * CHUNK

PROJ_BLK = 2000


def _proj_body(x_ref, w_ref, b_ref, o_ref):
    i = pl.program_id(0)
    x = x_ref[...]
    row = lax.broadcasted_iota(jnp.int32, (PROJ_BLK, 1), 0) + i * PROJ_BLK
    x = jnp.where(row == 0, 0.0, x)
    y = jnp.dot(x, w_ref[...], preferred_element_type=jnp.float32) + b_ref[...]
    half = PROJ_BLK // 2
    o_ref[...] = jnp.concatenate([y[:half], y[half:]], axis=1)


def _project(table, w, b):
    return pl.pallas_call(
        _proj_body,
        grid=(VOCAB // PROJ_BLK,),
        in_specs=[
            pl.BlockSpec((PROJ_BLK, EMB), lambda i: (i, 0)),
            pl.BlockSpec((EMB, OUT), lambda i: (0, 0)),
            pl.BlockSpec((1, OUT), lambda i: (0, 0)),
        ],
        out_specs=pl.BlockSpec((PROJ_BLK // 2, 2 * OUT), lambda i: (i, 0)),
        out_shape=jax.ShapeDtypeStruct((VOCAB // 2, 2 * OUT), jnp.float32),
    )(table, w, b.reshape(1, OUT))


DEPTH = 4


def _gather_body(p_hbm, idx_hbm, out_hbm, idx_v, *bufs):
    rows = bufs[:DEPTH]
    sg = bufs[DEPTH:]
    wid = lax.axis_index("s") * NC + lax.axis_index("c")
    pltpu.sync_copy(idx_hbm.at[wid], idx_v)
    base = wid * PER_W

    for b in range(DEPTH):
        pltpu.async_copy(p_hbm.at[idx_v.at[b]], rows[b], sg[b])

    def group(g, carry):
        for b in range(DEPTH):
            j = DEPTH * g + b

            @pl.when(j < N_CHUNK)
            def _():
                pltpu.make_async_copy(
                    p_hbm.at[idx_v.at[j]], rows[b], sg[b]
                ).wait()
                pltpu.sync_copy(
                    rows[b], out_hbm.at[pl.ds(base + j * CHUNK, CHUNK)]
                )
                nj = j + DEPTH

                @pl.when(nj < N_CHUNK)
                def _():
                    pltpu.async_copy(p_hbm.at[idx_v.at[nj]], rows[b], sg[b])
        return carry

    lax.fori_loop(0, (N_CHUNK + DEPTH - 1) // DEPTH, group, 0)


_gather = pl.kernel(
    _gather_body,
    out_type=jax.ShapeDtypeStruct((TOK_S, OUT), jnp.float32),
    mesh=plsc.VectorSubcoreMesh(core_axis_name="c", subcore_axis_name="s"),
    scratch_types=[pltpu.VMEM((N_CHUNK, CHUNK), jnp.int32)]
    + [pltpu.VMEM((CHUNK, OUT), jnp.float32) for _ in range(DEPTH)]
    + [pltpu.SemaphoreType.DMA for _ in range(DEPTH)],
    compiler_params=pltpu.CompilerParams(use_tc_tiling_on_sc=False),
)


def kernel(demo, emb_demo_weight, fc_weight, fc_bias):
    p = _project(
        emb_demo_weight.astype(jnp.float32),
        fc_weight.astype(jnp.float32),
        fc_bias.astype(jnp.float32),
    ).reshape(VOCAB, OUT)
    r = demo.astype(jnp.int32).reshape(NSLICE, NW, N_CHUNK, CHUNK)
    half = PROJ_BLK // 2
    rem = r % PROJ_BLK
    idx = (r - rem) + 2 * (rem % half) + rem // half
    parts = [
        _gather(p, idx[k]).reshape(B // NSLICE, F * OUT)
        for k in range(NSLICE)
    ]
    return jnp.concatenate(parts, axis=0)


# speedup vs baseline: 1.3007x; 1.3007x over previous
import jax
import jax.numpy as jnp
from jax import lax
from jax.experimental import pallas as pl
from jax.experimental.pallas import tpu as pltpu
from jax.experimental.pallas import tpu_sc as plsc

VOCAB = 100000
EMB = 128
OUT = 64
B = 4096
F = 26

NC, NS = 2, 16
NW = NC * NS
TOK = B * F
CHUNK = 128
N_CHUNK = TOK // (NW * CHUNK)
PER_W = N_CHUNK * CHUNK

PROJ_BLK = 4000


def _proj_body(x_ref, w_ref, b_ref, o_ref):
    i = pl.program_id(0)
    x = x_ref[...]
    row = lax.broadcasted_iota(jnp.int32, (PROJ_BLK, 1), 0) + i * PROJ_BLK
    x = jnp.where(row == 0, 0.0, x)
    y = jnp.dot(x, w_ref[...], preferred_element_type=jnp.float32) + b_ref[...]
    half = PROJ_BLK // 2
    o_ref[...] = jnp.concatenate([y[:half], y[half:]], axis=1)


def _project(table, w, b):
    return pl.pallas_call(
        _proj_body,
        grid=(VOCAB // PROJ_BLK,),
        in_specs=[
            pl.BlockSpec((PROJ_BLK, EMB), lambda i: (i, 0)),
            pl.BlockSpec((EMB, OUT), lambda i: (0, 0)),
            pl.BlockSpec((1, OUT), lambda i: (0, 0)),
        ],
        out_specs=pl.BlockSpec((PROJ_BLK // 2, 2 * OUT), lambda i: (i, 0)),
        out_shape=jax.ShapeDtypeStruct((VOCAB // 2, 2 * OUT), jnp.float32),
    )(table, w, b.reshape(1, OUT))


DEPTH = 4


def _gather_body(p_hbm, idx_hbm, out_hbm, idx_v, *bufs):
    rows = bufs[:DEPTH]
    sg = bufs[DEPTH:]
    wid = lax.axis_index("s") * NC + lax.axis_index("c")
    pltpu.sync_copy(idx_hbm.at[wid], idx_v)
    base = wid * PER_W

    for b in range(DEPTH):
        pltpu.async_copy(p_hbm.at[idx_v.at[b]], rows[b], sg[b])

    def group(g, carry):
        for b in range(DEPTH):
            j = DEPTH * g + b

            @pl.when(j < N_CHUNK)
            def _():
                pltpu.make_async_copy(
                    p_hbm.at[idx_v.at[j]], rows[b], sg[b]
                ).wait()
                pltpu.sync_copy(
                    rows[b], out_hbm.at[pl.ds(base + j * CHUNK, CHUNK)]
                )
                nj = j + DEPTH

                @pl.when(nj < N_CHUNK)
                def _():
                    pltpu.async_copy(p_hbm.at[idx_v.at[nj]], rows[b], sg[b])
        return carry

    lax.fori_loop(0, (N_CHUNK + DEPTH - 1) // DEPTH, group, 0)


_gather = pl.kernel(
    _gather_body,
    out_type=jax.ShapeDtypeStruct((TOK, OUT), jnp.float32),
    mesh=plsc.VectorSubcoreMesh(core_axis_name="c", subcore_axis_name="s"),
    scratch_types=[pltpu.VMEM((N_CHUNK, CHUNK), jnp.int32)]
    + [pltpu.VMEM((CHUNK, OUT), jnp.float32) for _ in range(DEPTH)]
    + [pltpu.SemaphoreType.DMA for _ in range(DEPTH)],
    compiler_params=pltpu.CompilerParams(use_tc_tiling_on_sc=False),
)


def kernel(demo, emb_demo_weight, fc_weight, fc_bias):
    p = _project(
        emb_demo_weight.astype(jnp.float32),
        fc_weight.astype(jnp.float32),
        fc_bias.astype(jnp.float32),
    ).reshape(VOCAB, OUT)
    r = demo.astype(jnp.int32).reshape(NW, N_CHUNK, CHUNK)
    half = PROJ_BLK // 2
    rem = r % PROJ_BLK
    idx = (r - rem) + 2 * (rem % half) + rem // half
    return _gather(p, idx).reshape(B, F * OUT)
